# baseline (speedup 1.0000x reference)
import functools
import math

import jax
import jax.numpy as jnp
from jax import lax
from jax.experimental import pallas as pl
from jax.experimental.pallas import tpu as pltpu
from jax.experimental.pallas import tpu_sc as plsc

_BETA = 0.99
_C = 1000
_B = 16384
_ROWS = 4096
_GRID = _B // _ROWS
_LN_BETA = math.log(_BETA)

_NS = 16
_CHUNK = _B // _NS
_CPAD = 1024
_WS = _CPAD // _NS
_L = 16


def _sc_body(t_hbm, w_hbm, t_v, t2_v, ones_v, z_v, cs_v, ws_v, shared_cnt):
    sid = lax.axis_index("s")
    base = sid * _CHUNK

    pltpu.sync_copy(t_hbm.at[pl.ds(base, _CHUNK)], t_v)

    def _fill(j, c):
        ones_v[pl.ds(j * _L, _L)] = jnp.ones((_L,), jnp.float32)
        t2_v[j // 8, pl.ds((j % 8) * _L, _L)] = t_v[pl.ds(j * _L, _L)]
        return c
    lax.fori_loop(0, _CHUNK // _L, _fill, 0)

    @pl.when(sid == 0)
    def _zero():
        def _z(j, c):
            z_v[pl.ds(j * _L, _L)] = jnp.zeros((_L,), jnp.float32)
            return c
        lax.fori_loop(0, _CPAD // _L, _z, 0)

    plsc.subcore_barrier()

    @pl.when(sid == 0)
    def _zcopy():
        pltpu.sync_copy(z_v, shared_cnt)

    plsc.subcore_barrier()

    def _hist(j, c):
        pltpu.sync_copy(ones_v.at[pl.ds(j * 128, 128)],
                        shared_cnt.at[t2_v.at[j]], add=True)
        return c
    lax.fori_loop(0, _CHUNK // 128, _hist, 0)

    plsc.subcore_barrier()
    pltpu.sync_copy(shared_cnt.at[pl.ds(sid * _WS, _WS)], cs_v)

    def _weights(j, c):
        cnt = cs_v[pl.ds(j * _L, _L)]
        freq = cnt * (1.0 / _B)
        eff = 1.0 - jnp.exp(freq * _LN_BETA)
        ws_v[pl.ds(j * _L, _L)] = (1.0 - _BETA) / eff
        return c
    lax.fori_loop(0, _WS // _L, _weights, 0)

    plsc.subcore_barrier()
    pltpu.sync_copy(ws_v, w_hbm.at[0, pl.ds(sid * _WS, _WS)])


_sc_hist_kernel = functools.partial(
    pl.kernel,
    out_type=jax.ShapeDtypeStruct((1, _CPAD), jnp.float32),
    mesh=plsc.VectorSubcoreMesh(core_axis_name="c", subcore_axis_name="s",
                                num_cores=1),
    compiler_params=pltpu.CompilerParams(needs_layout_passes=False),
    scratch_types=[
        pltpu.VMEM((_CHUNK,), jnp.int32),
        pltpu.VMEM((_CHUNK // 128, 128), jnp.int32),
        pltpu.VMEM((_CHUNK,), jnp.float32),
        pltpu.VMEM((_CPAD,), jnp.float32),
        pltpu.VMEM((_WS,), jnp.float32),
        pltpu.VMEM((_WS,), jnp.float32),
        pltpu.VMEM_SHARED((_CPAD,), jnp.float32),
    ],
)(_sc_body)


def _tc_body(x_ref, t_ref, w_ref, loss_ref, num_ref, den_ref):
    i = pl.program_id(0)

    @pl.when(i == 0)
    def _init():
        num_ref[...] = jnp.zeros_like(num_ref)
        den_ref[...] = jnp.zeros_like(den_ref)

    x = x_ref[...]
    t = t_ref[0, 0, :]
    m = jnp.max(x, axis=1, keepdims=True)
    e = jnp.exp(x - m)
    se = jnp.sum(e, axis=1, keepdims=True)
    cols = jax.lax.broadcasted_iota(jnp.int32, (_ROWS, _C), 1)
    mask = cols == t[:, None]
    picked = jnp.sum(jnp.where(mask, x, 0.0), axis=1, keepdims=True)
    wb = w_ref[0:1, 0:_C]
    wrow = jnp.sum(jnp.where(mask, wb, 0.0), axis=1, keepdims=True)
    nll = m + jnp.log(se) - picked
    num_ref[...] += jnp.sum(wrow * nll)[None, None]
    den_ref[...] += jnp.sum(wrow)[None, None]

    @pl.when(i == _GRID - 1)
    def _fin():
        loss_ref[...] = num_ref[...] / den_ref[...]


def kernel(output, target):
    w = _sc_hist_kernel(target.astype(jnp.int32))
    t3 = target.astype(jnp.int32).reshape(_GRID, 1, _ROWS)
    loss = pl.pallas_call(
        _tc_body,
        grid=(_GRID,),
        in_specs=[
            pl.BlockSpec((_ROWS, _C), lambda i: (i, 0)),
            pl.BlockSpec((1, 1, _ROWS), lambda i: (i, 0, 0)),
            pl.BlockSpec((1, _CPAD), lambda i: (0, 0)),
        ],
        out_specs=pl.BlockSpec((1, 1), lambda i: (0, 0)),
        out_shape=jax.ShapeDtypeStruct((1, 1), jnp.float32),
        scratch_shapes=[
            pltpu.VMEM((1, 1), jnp.float32),
            pltpu.VMEM((1, 1), jnp.float32),
        ],
    )(output, t3, w)
    return loss[0, 0]


# speedup vs baseline: 1.0389x; 1.0389x over previous
import functools
import math

import jax
import jax.numpy as jnp
from jax import lax
from jax.experimental import pallas as pl
from jax.experimental.pallas import tpu as pltpu
from jax.experimental.pallas import tpu_sc as plsc

_BETA = 0.99
_C = 1000
_B = 16384
_ROWS = 2048
_GRID = _B // _ROWS
_LN_BETA = math.log(_BETA)

_NS = 16
_CHUNK = _B // _NS
_CPAD = 1024
_WS = _CPAD // _NS
_L = 16


def _sc_body(t_hbm, w_hbm, t_v, t2_v, ones_v, z_v, cs_v, ws_v, shared_cnt):
    sid = lax.axis_index("s")
    base = sid * _CHUNK

    pltpu.sync_copy(t_hbm.at[pl.ds(base, _CHUNK)], t_v)

    def _fill(j, c):
        ones_v[pl.ds(j * _L, _L)] = jnp.ones((_L,), jnp.float32)
        t2_v[j // 8, pl.ds((j % 8) * _L, _L)] = t_v[pl.ds(j * _L, _L)]
        return c
    lax.fori_loop(0, _CHUNK // _L, _fill, 0)

    @pl.when(sid == 0)
    def _zero():
        def _z(j, c):
            z_v[pl.ds(j * _L, _L)] = jnp.zeros((_L,), jnp.float32)
            return c
        lax.fori_loop(0, _CPAD // _L, _z, 0)

    plsc.subcore_barrier()

    @pl.when(sid == 0)
    def _zcopy():
        pltpu.sync_copy(z_v, shared_cnt)

    plsc.subcore_barrier()

    def _hist(j, c):
        pltpu.sync_copy(ones_v.at[pl.ds(j * 128, 128)],
                        shared_cnt.at[t2_v.at[j]], add=True)
        return c
    lax.fori_loop(0, _CHUNK // 128, _hist, 0)

    plsc.subcore_barrier()
    pltpu.sync_copy(shared_cnt.at[pl.ds(sid * _WS, _WS)], cs_v)

    def _weights(j, c):
        cnt = cs_v[pl.ds(j * _L, _L)]
        freq = cnt * (1.0 / _B)
        eff = 1.0 - jnp.exp(freq * _LN_BETA)
        ws_v[pl.ds(j * _L, _L)] = (1.0 - _BETA) / eff
        return c
    lax.fori_loop(0, _WS // _L, _weights, 0)

    plsc.subcore_barrier()
    pltpu.sync_copy(ws_v, w_hbm.at[0, pl.ds(sid * _WS, _WS)])


_sc_hist_kernel = functools.partial(
    pl.kernel,
    out_type=jax.ShapeDtypeStruct((1, _CPAD), jnp.float32),
    mesh=plsc.VectorSubcoreMesh(core_axis_name="c", subcore_axis_name="s",
                                num_cores=1),
    compiler_params=pltpu.CompilerParams(needs_layout_passes=False),
    scratch_types=[
        pltpu.VMEM((_CHUNK,), jnp.int32),
        pltpu.VMEM((_CHUNK // 128, 128), jnp.int32),
        pltpu.VMEM((_CHUNK,), jnp.float32),
        pltpu.VMEM((_CPAD,), jnp.float32),
        pltpu.VMEM((_WS,), jnp.float32),
        pltpu.VMEM((_WS,), jnp.float32),
        pltpu.VMEM_SHARED((_CPAD,), jnp.float32),
    ],
)(_sc_body)


def _tc_body(x_ref, t_ref, w_ref, loss_ref, num_ref, den_ref):
    i = pl.program_id(0)

    @pl.when(i == 0)
    def _init():
        num_ref[...] = jnp.zeros_like(num_ref)
        den_ref[...] = jnp.zeros_like(den_ref)

    x = x_ref[...]
    t = t_ref[0, 0, :]
    m = jnp.max(x, axis=1, keepdims=True)
    e = jnp.exp(x - m)
    se = jnp.sum(e, axis=1, keepdims=True)
    cols = jax.lax.broadcasted_iota(jnp.int32, (_ROWS, _C), 1)
    mask = cols == t[:, None]
    picked = jnp.sum(jnp.where(mask, x, 0.0), axis=1, keepdims=True)
    wb = w_ref[0:1, 0:_C]
    wrow = jnp.sum(jnp.where(mask, wb, 0.0), axis=1, keepdims=True)
    nll = m + jnp.log(se) - picked
    num_ref[...] += jnp.sum(wrow * nll)[None, None]
    den_ref[...] += jnp.sum(wrow)[None, None]

    @pl.when(i == _GRID - 1)
    def _fin():
        loss_ref[...] = num_ref[...] / den_ref[...]


def kernel(output, target):
    w = _sc_hist_kernel(target.astype(jnp.int32))
    t3 = target.astype(jnp.int32).reshape(_GRID, 1, _ROWS)
    loss = pl.pallas_call(
        _tc_body,
        grid=(_GRID,),
        in_specs=[
            pl.BlockSpec((_ROWS, _C), lambda i: (i, 0)),
            pl.BlockSpec((1, 1, _ROWS), lambda i: (i, 0, 0)),
            pl.BlockSpec((1, _CPAD), lambda i: (0, 0)),
        ],
        out_specs=pl.BlockSpec((1, 1), lambda i: (0, 0)),
        out_shape=jax.ShapeDtypeStruct((1, 1), jnp.float32),
        scratch_shapes=[
            pltpu.VMEM((1, 1), jnp.float32),
            pltpu.VMEM((1, 1), jnp.float32),
        ],
    )(output, t3, w)
    return loss[0, 0]


# speedup vs baseline: 1.1298x; 1.0875x over previous
import functools
import math

import jax
import jax.numpy as jnp
from jax import lax
from jax.experimental import pallas as pl
from jax.experimental.pallas import tpu as pltpu
from jax.experimental.pallas import tpu_sc as plsc

_BETA = 0.99
_C = 1000
_B = 16384
_ROWS = 2048
_GRID = _B // _ROWS
_LN_BETA = math.log(_BETA)

_NS = 16
_CHUNK = _B // _NS
_CPAD = 1024
_L = 16


def _sc_body(t_hbm, wt_hbm, t_v, t2_v, ones_v, z_v, cnt_v, w_v, wt_v,
             shared_cnt):
    sid = lax.axis_index("s")
    base = sid * _CHUNK

    pltpu.sync_copy(t_hbm.at[pl.ds(base, _CHUNK)], t_v)

    def _fill(j, c):
        ones_v[pl.ds(j * _L, _L)] = jnp.ones((_L,), jnp.float32)
        t2_v[j // 8, pl.ds((j % 8) * _L, _L)] = t_v[pl.ds(j * _L, _L)]
        return c
    lax.fori_loop(0, _CHUNK // _L, _fill, 0)

    @pl.when(sid == 0)
    def _zero():
        def _z(j, c):
            z_v[pl.ds(j * _L, _L)] = jnp.zeros((_L,), jnp.float32)
            return c
        lax.fori_loop(0, _CPAD // _L, _z, 0)

    plsc.subcore_barrier()

    @pl.when(sid == 0)
    def _zcopy():
        pltpu.sync_copy(z_v, shared_cnt)

    plsc.subcore_barrier()

    def _hist(j, c):
        pltpu.sync_copy(ones_v.at[pl.ds(j * 128, 128)],
                        shared_cnt.at[t2_v.at[j]], add=True)
        return c
    lax.fori_loop(0, _CHUNK // 128, _hist, 0)

    plsc.subcore_barrier()
    pltpu.sync_copy(shared_cnt, cnt_v)

    def _weights(j, c):
        cnt = cnt_v[pl.ds(j * _L, _L)]
        freq = cnt * (1.0 / _B)
        eff = 1.0 - jnp.exp(freq * _LN_BETA)
        w_v[pl.ds(j * _L, _L)] = (1.0 - _BETA) / eff
        return c
    lax.fori_loop(0, _CPAD // _L, _weights, 0)

    def _gather(j, c):
        ts = t_v[pl.ds(j * _L, _L)]
        wt_v[pl.ds(j * _L, _L)] = plsc.load_gather(w_v, [ts])
        return c
    lax.fori_loop(0, _CHUNK // _L, _gather, 0)

    plsc.subcore_barrier()
    nsub = _ROWS // _CHUNK
    pltpu.sync_copy(
        wt_v, wt_hbm.at[sid // nsub, 0,
                        pl.ds((sid % nsub) * _CHUNK, _CHUNK)])


_sc_weight_kernel = functools.partial(
    pl.kernel,
    out_type=jax.ShapeDtypeStruct((_GRID, 1, _ROWS), jnp.float32),
    mesh=plsc.VectorSubcoreMesh(core_axis_name="c", subcore_axis_name="s",
                                num_cores=1),
    compiler_params=pltpu.CompilerParams(needs_layout_passes=False),
    scratch_types=[
        pltpu.VMEM((_CHUNK,), jnp.int32),
        pltpu.VMEM((_CHUNK // 128, 128), jnp.int32),
        pltpu.VMEM((_CHUNK,), jnp.float32),
        pltpu.VMEM((_CPAD,), jnp.float32),
        pltpu.VMEM((_CPAD,), jnp.float32),
        pltpu.VMEM((_CPAD,), jnp.float32),
        pltpu.VMEM((_CHUNK,), jnp.float32),
        pltpu.VMEM_SHARED((_CPAD,), jnp.float32),
    ],
)(_sc_body)


def _tc_body(x_ref, t_ref, wt_ref, loss_ref, num_ref, den_ref):
    i = pl.program_id(0)

    @pl.when(i == 0)
    def _init():
        num_ref[...] = jnp.zeros_like(num_ref)
        den_ref[...] = jnp.zeros_like(den_ref)

    x = x_ref[...]
    t = t_ref[0, 0, :]
    wt = wt_ref[0, 0, :][None, :]
    m = jnp.max(x, axis=1, keepdims=True)
    e = jnp.exp(x - m)
    se = jnp.sum(e, axis=1, keepdims=True)
    cols = jax.lax.broadcasted_iota(jnp.int32, (_ROWS, _C), 1)
    mask = cols == t[:, None]
    picked = jnp.sum(jnp.where(mask, x, 0.0), axis=1, keepdims=True)
    nll = (m + jnp.log(se) - picked).T
    num_ref[...] += jnp.sum(wt * nll)[None, None]
    den_ref[...] += jnp.sum(wt)[None, None]

    @pl.when(i == _GRID - 1)
    def _fin():
        loss_ref[...] = num_ref[...] / den_ref[...]


def kernel(output, target):
    wt = _sc_weight_kernel(target.astype(jnp.int32))
    t3 = target.astype(jnp.int32).reshape(_GRID, 1, _ROWS)
    loss = pl.pallas_call(
        _tc_body,
        grid=(_GRID,),
        in_specs=[
            pl.BlockSpec((_ROWS, _C), lambda i: (i, 0)),
            pl.BlockSpec((1, 1, _ROWS), lambda i: (i, 0, 0)),
            pl.BlockSpec((1, 1, _ROWS), lambda i: (i, 0, 0)),
        ],
        out_specs=pl.BlockSpec((1, 1), lambda i: (0, 0)),
        out_shape=jax.ShapeDtypeStruct((1, 1), jnp.float32),
        scratch_shapes=[
            pltpu.VMEM((1, 1), jnp.float32),
            pltpu.VMEM((1, 1), jnp.float32),
        ],
    )(output, t3, wt)
    return loss[0, 0]
